# baseline (speedup 1.0000x reference)
import functools

import jax
import jax.numpy as jnp
from jax import lax
from jax.experimental import pallas as pl
from jax.experimental.pallas import tpu as pltpu
from jax.experimental.pallas import tpu_sc as plsc

E = 8
K = 2
H = 1024
I = 4096
T = 2048
TILE = 256
G = 24
S = G * TILE
TILE_I = 1024
CHUNK = 256

NC = 2
NS = 16
NW = NC * NS
RPT = S // NW
TPT = T // NW


def _gelu_erf(x):
    return 0.5 * x * (1.0 + lax.erf(x * 0.7071067811865476))


def _router_body(x_ref, wr_ref, sp_ref, pos1_ref, pos2_ref, wn1_ref,
                 wn2_ref, aux_ref):
    x = x_ref[...]
    wr = wr_ref[...]
    logits = lax.dot_general(x, wr, (((1,), (1,)), ((), ())),
                             preferred_element_type=jnp.float32)
    m = jnp.max(logits, axis=1, keepdims=True)
    ex = jnp.exp(logits - m)
    probs = ex / jnp.sum(ex, axis=1, keepdims=True)

    iota = lax.broadcasted_iota(jnp.int32, (T, E), 1)
    m1 = jnp.max(probs, axis=1, keepdims=True)
    i1 = jnp.min(jnp.where(probs == m1, iota, E), axis=1, keepdims=True)
    p2 = jnp.where(iota == i1, -1.0, probs)
    m2 = jnp.max(p2, axis=1, keepdims=True)
    i2 = jnp.min(jnp.where(p2 == m2, iota, E), axis=1, keepdims=True)
    ssum = m1 + m2
    hot1 = (iota == i1).astype(jnp.float32)
    hot2 = (iota == i2).astype(jnp.float32)
    wn1_ref[...] = m1 / ssum
    wn2_ref[...] = m2 / ssum

    tri = (lax.broadcasted_iota(jnp.int32, (CHUNK, CHUNK), 0)
           > lax.broadcasted_iota(jnp.int32, (CHUNK, CHUNK), 1)
           ).astype(jnp.bfloat16)
    carry = jnp.zeros((1, E), jnp.float32)
    ranks = []
    for hot in (hot1, hot2):
        hot_b = hot.astype(jnp.bfloat16)
        rk = []
        for c in range(T // CHUNK):
            hc = hot[c * CHUNK:(c + 1) * CHUNK]
            hcb = hot_b[c * CHUNK:(c + 1) * CHUNK]
            cum = lax.dot_general(tri, hcb, (((1,), (0,)), ((), ())),
                                  preferred_element_type=jnp.float32) + carry
            rk.append(jnp.sum(cum * hc, axis=1, keepdims=True))
            carry = carry + jnp.sum(hc, axis=0, keepdims=True)
        ranks.append(jnp.concatenate(rk, axis=0))
    rank1, rank2 = ranks
    counts_row = carry

    hotsum = hot1 + hot2
    ones_t = jnp.ones((T, 1), jnp.float32)
    counts_col = lax.dot_general(hotsum, ones_t, (((0,), (0,)), ((), ())),
                                 preferred_element_type=jnp.float32)
    ntiles_col = jnp.floor((counts_col + (TILE - 1)) * (1.0 / TILE))
    ltri8 = (lax.broadcasted_iota(jnp.int32, (E, E), 0)
             > lax.broadcasted_iota(jnp.int32, (E, E), 1)).astype(jnp.float32)
    segt_col = lax.dot_general(ltri8, ntiles_col, (((1,), (0,)), ((), ())),
                               preferred_element_type=jnp.float32)

    seg1 = lax.dot_general(hot1, segt_col, (((1,), (0,)), ((), ())),
                           preferred_element_type=jnp.float32)
    seg2 = lax.dot_general(hot2, segt_col, (((1,), (0,)), ((), ())),
                           preferred_element_type=jnp.float32)
    pos1_ref[...] = (rank1 + TILE * seg1).astype(jnp.int32)
    pos2_ref[...] = (rank2 + TILE * seg2).astype(jnp.int32)

    gi = lax.broadcasted_iota(jnp.int32, (E, G), 1).astype(jnp.float32)
    ei = lax.broadcasted_iota(jnp.int32, (E, G), 0).astype(jnp.float32)
    covg = jnp.logical_and(segt_col <= gi, gi < segt_col + ntiles_col
                           ).astype(jnp.float32)
    texp = jnp.sum(ei * covg, axis=0, keepdims=True)
    cover = jnp.sum(covg, axis=0, keepdims=True)
    le = jnp.max((lax.broadcasted_iota(jnp.int32, (E, 1), 0
                                       ).astype(jnp.float32) + 1.0)
                 * (ntiles_col > 0), axis=0, keepdims=True) - 1.0
    texp = texp + (1.0 - cover) * le
    total = jnp.sum(ntiles_col, axis=0, keepdims=True)
    sp_ref[...] = jnp.concatenate(
        [total.astype(jnp.int32), texp.astype(jnp.int32)], axis=1)

    f = counts_row * (1.0 / (T * K))
    p_mean = jnp.mean(probs, axis=0, keepdims=True)
    aux_ref[...] = E * jnp.sum(f * p_mean, keepdims=True).reshape(1, 1)


IR = 16
IC_ = T // IR
NCH = 4
RCH = RPT // NCH


def _dispatch_body(x_hbm, pos1_hbm, pos2_hbm, wn1_hbm, wn2_hbm,
                   xs_hbm, sw_hbm,
                   p1row, p2row, w1row, w2row, tokrow, zrow, myidx,
                   rows_a, rows_b, shidx, gsa, gsb, wsa, wsb):
    c = lax.axis_index("c")
    s = lax.axis_index("s")
    wid = c * NS + s

    zi = jnp.zeros((16,), jnp.int32)

    def zero(i, acc):
        zrow[pl.ds(i * 16, 16)] = zi
        return acc
    lax.fori_loop(0, (S // NS) // 16, zero, 0)
    pltpu.sync_copy(zrow, shidx.at[pl.ds(s * (S // NS), S // NS)])
    pltpu.sync_copy(pos1_hbm.at[s], p1row)
    pltpu.sync_copy(pos2_hbm.at[s], p2row)
    pltpu.sync_copy(wn1_hbm.at[s], w1row)
    pltpu.sync_copy(wn2_hbm.at[s], w2row)

    def mktok(i, acc):
        tokrow[pl.ds(i * 16, 16)] = (lax.iota(jnp.int32, 16)
                                     + (s * IC_ + i * 16))
        return acc
    lax.fori_loop(0, IC_ // 16, mktok, 0)
    plsc.subcore_barrier()
    descs = [
        pltpu.async_copy(tokrow, shidx.at[p1row], gsa),
        pltpu.async_copy(tokrow, shidx.at[p2row], gsb),
        pltpu.async_copy(w1row, sw_hbm.at[p1row], wsa),
        pltpu.async_copy(w2row, sw_hbm.at[p2row], wsb),
    ]
    for d in descs:
        d.wait()
    plsc.subcore_barrier()

    pltpu.sync_copy(shidx.at[pl.ds(wid * RPT, RPT)], myidx)
    bufs = (rows_a, rows_b)
    gsems = (gsa, gsb)
    wsems = (wsa, wsb)

    def gidx(ch):
        return myidx.at[pl.ds(ch * RCH, RCH)]

    gd = [None] * NCH
    wd = [None] * NCH
    gd[0] = pltpu.async_copy(x_hbm.at[gidx(0)], bufs[0], gsems[0])
    for ch in range(NCH):
        cur = ch % 2
        if ch + 1 < NCH:
            if ch >= 1:
                wd[ch - 1].wait()
            gd[ch + 1] = pltpu.async_copy(
                x_hbm.at[gidx(ch + 1)], bufs[1 - cur], gsems[1 - cur])
        gd[ch].wait()
        wd[ch] = pltpu.async_copy(
            bufs[cur], xs_hbm.at[pl.ds(wid * RPT + ch * RCH, RCH)],
            wsems[cur])
    wd[NCH - 2].wait()
    wd[NCH - 1].wait()


def _grouped_body(sp_ref, x_ref, wu_ref, wd_ref, w_ref, o_ref):
    ic = pl.program_id(0)
    g = pl.program_id(1)
    valid = g < sp_ref[0]

    @pl.when(valid)
    def _compute():
        xb = x_ref[...].astype(jnp.bfloat16)
        h = lax.dot_general(xb, wu_ref[0].astype(jnp.bfloat16),
                            (((1,), (1,)), ((), ())),
                            preferred_element_type=jnp.float32)
        h = _gelu_erf(h)
        y = lax.dot_general(h.astype(jnp.bfloat16),
                            wd_ref[0].astype(jnp.bfloat16),
                            (((1,), (1,)), ((), ())),
                            preferred_element_type=jnp.float32)
        y = w_ref[...] * y

        @pl.when(ic == 0)
        def _set():
            o_ref[pl.ds(g * TILE, TILE), :] = y

        @pl.when(ic > 0)
        def _acc():
            o_ref[pl.ds(g * TILE, TILE), :] += y


def _combine_body(y_hbm, pos1_hbm, pos2_hbm, out_hbm,
                  i1v, i2v, r1, r2, sem):
    c = lax.axis_index("c")
    s = lax.axis_index("s")
    wid = c * NS + s
    base = wid * TPT
    pltpu.sync_copy(pos1_hbm.at[pl.ds(base, TPT)], i1v)
    pltpu.sync_copy(pos2_hbm.at[pl.ds(base, TPT)], i2v)
    half = TPT // 2
    for ch in range(2):
        a = pltpu.async_copy(y_hbm.at[i1v.at[pl.ds(ch * half, half)]],
                             r1, sem)
        b = pltpu.async_copy(y_hbm.at[i2v.at[pl.ds(ch * half, half)]],
                             r2, sem)
        a.wait()
        b.wait()

        def add_tok(t, acc):
            for j in range(H // 16):
                sl = pl.ds(j * 16, 16)
                r1[t, sl] = r1[t, sl] + r2[t, sl]
            return acc
        lax.fori_loop(0, half, add_tok, 0)
        pltpu.sync_copy(r1, out_hbm.at[pl.ds(base + ch * half, half)])


@jax.jit
def kernel(hidden_states, W_router, W_up, W_down):
    batch, seq_len, hidden = hidden_states.shape
    flat = hidden_states.reshape(-1, hidden)

    sp, pos1, pos2, wn1, wn2, aux = pl.pallas_call(
        _router_body,
        out_shape=[
            jax.ShapeDtypeStruct((1, G + 1), jnp.int32),
            jax.ShapeDtypeStruct((T, 1), jnp.int32),
            jax.ShapeDtypeStruct((T, 1), jnp.int32),
            jax.ShapeDtypeStruct((T, 1), jnp.float32),
            jax.ShapeDtypeStruct((T, 1), jnp.float32),
            jax.ShapeDtypeStruct((1, 1), jnp.float32),
        ],
    )(flat, W_router)

    pos1f = pos1.reshape(T)
    pos2f = pos2.reshape(T)

    mesh = plsc.VectorSubcoreMesh(core_axis_name="c", subcore_axis_name="s")
    x_sorted, slot_w = pl.kernel(
        _dispatch_body,
        out_type=[
            jax.ShapeDtypeStruct((S, H), jnp.float32),
            jax.ShapeDtypeStruct((S,), jnp.float32),
        ],
        mesh=mesh,
        scratch_types=[
            pltpu.VMEM((IC_,), jnp.int32),
            pltpu.VMEM((IC_,), jnp.int32),
            pltpu.VMEM((IC_,), jnp.float32),
            pltpu.VMEM((IC_,), jnp.float32),
            pltpu.VMEM((IC_,), jnp.int32),
            pltpu.VMEM((S // NS,), jnp.int32),
            pltpu.VMEM((RPT,), jnp.int32),
            pltpu.VMEM((RCH, H), jnp.float32),
            pltpu.VMEM((RCH, H), jnp.float32),
            pltpu.VMEM_SHARED((S,), jnp.int32),
            pltpu.SemaphoreType.DMA,
            pltpu.SemaphoreType.DMA,
            pltpu.SemaphoreType.DMA,
            pltpu.SemaphoreType.DMA,
        ],
    )(flat, pos1f.reshape(IR, IC_), pos2f.reshape(IR, IC_),
      wn1.reshape(IR, IC_), wn2.reshape(IR, IC_))

    y_sorted = pl.pallas_call(
        _grouped_body,
        grid_spec=pltpu.PrefetchScalarGridSpec(
            num_scalar_prefetch=1,
            grid=(I // TILE_I, G),
            in_specs=[
                pl.BlockSpec((TILE, H), lambda ic, g, sp: (g, 0)),
                pl.BlockSpec((1, TILE_I, H), lambda ic, g, sp: (sp[1 + g], ic, 0)),
                pl.BlockSpec((1, H, TILE_I), lambda ic, g, sp: (sp[1 + g], 0, ic)),
                pl.BlockSpec((TILE, 1), lambda ic, g, sp: (g, 0)),
            ],
            out_specs=pl.BlockSpec((S, H), lambda ic, g, sp: (0, 0)),
        ),
        out_shape=jax.ShapeDtypeStruct((S, H), jnp.float32),
        compiler_params=pltpu.CompilerParams(
            dimension_semantics=("arbitrary", "arbitrary")),
    )(sp.reshape(G + 1), x_sorted, W_up, W_down, slot_w.reshape(S, 1))

    out = pl.kernel(
        _combine_body,
        out_type=jax.ShapeDtypeStruct((T, H), jnp.float32),
        mesh=plsc.VectorSubcoreMesh(core_axis_name="c", subcore_axis_name="s"),
        scratch_types=[
            pltpu.VMEM((TPT,), jnp.int32),
            pltpu.VMEM((TPT,), jnp.int32),
            pltpu.VMEM((TPT // 2, H), jnp.float32),
            pltpu.VMEM((TPT // 2, H), jnp.float32),
            pltpu.SemaphoreType.DMA,
        ],
    )(y_sorted, pos1f, pos2f)

    return out.reshape(batch, seq_len, hidden), aux.reshape(())


# speedup vs baseline: 1.3147x; 1.3147x over previous
import functools

import jax
import jax.numpy as jnp
from jax import lax
from jax.experimental import pallas as pl
from jax.experimental.pallas import tpu as pltpu
from jax.experimental.pallas import tpu_sc as plsc

E = 8
K = 2
H = 1024
I = 4096
T = 2048
TILE = 256
G = 24
S = G * TILE
TILE_I = 2048
HP = H // 2
CHUNK = 256

NC = 2
NS = 16
NW = NC * NS
RPT = S // NW
TPT = T // NW


def _gelu_erf(x):
    return 0.5 * x * (1.0 + lax.erf(x * 0.7071067811865476))


def _router_body(x_ref, wr_ref, sp_ref, pos1_ref, pos2_ref, wn1_ref,
                 wn2_ref, aux_ref):
    x = x_ref[...]
    wr = wr_ref[...]
    logits = lax.dot_general(x, wr, (((1,), (1,)), ((), ())),
                             preferred_element_type=jnp.float32)
    m = jnp.max(logits, axis=1, keepdims=True)
    ex = jnp.exp(logits - m)
    probs = ex / jnp.sum(ex, axis=1, keepdims=True)

    iota = lax.broadcasted_iota(jnp.int32, (T, E), 1)
    m1 = jnp.max(probs, axis=1, keepdims=True)
    i1 = jnp.min(jnp.where(probs == m1, iota, E), axis=1, keepdims=True)
    p2 = jnp.where(iota == i1, -1.0, probs)
    m2 = jnp.max(p2, axis=1, keepdims=True)
    i2 = jnp.min(jnp.where(p2 == m2, iota, E), axis=1, keepdims=True)
    ssum = m1 + m2
    hot1 = (iota == i1).astype(jnp.float32)
    hot2 = (iota == i2).astype(jnp.float32)
    wn1_ref[...] = m1 / ssum
    wn2_ref[...] = m2 / ssum

    tri = (lax.broadcasted_iota(jnp.int32, (CHUNK, CHUNK), 0)
           > lax.broadcasted_iota(jnp.int32, (CHUNK, CHUNK), 1)
           ).astype(jnp.bfloat16)
    carry = jnp.zeros((1, E), jnp.float32)
    ranks = []
    for hot in (hot1, hot2):
        hot_b = hot.astype(jnp.bfloat16)
        rk = []
        for c in range(T // CHUNK):
            hc = hot[c * CHUNK:(c + 1) * CHUNK]
            hcb = hot_b[c * CHUNK:(c + 1) * CHUNK]
            cum = lax.dot_general(tri, hcb, (((1,), (0,)), ((), ())),
                                  preferred_element_type=jnp.float32) + carry
            rk.append(jnp.sum(cum * hc, axis=1, keepdims=True))
            carry = carry + jnp.sum(hc, axis=0, keepdims=True)
        ranks.append(jnp.concatenate(rk, axis=0))
    rank1, rank2 = ranks
    counts_row = carry

    hotsum = hot1 + hot2
    ones_t = jnp.ones((T, 1), jnp.float32)
    counts_col = lax.dot_general(hotsum, ones_t, (((0,), (0,)), ((), ())),
                                 preferred_element_type=jnp.float32)
    ntiles_col = jnp.floor((counts_col + (TILE - 1)) * (1.0 / TILE))
    ltri8 = (lax.broadcasted_iota(jnp.int32, (E, E), 0)
             > lax.broadcasted_iota(jnp.int32, (E, E), 1)).astype(jnp.float32)
    segt_col = lax.dot_general(ltri8, ntiles_col, (((1,), (0,)), ((), ())),
                               preferred_element_type=jnp.float32)

    seg1 = lax.dot_general(hot1, segt_col, (((1,), (0,)), ((), ())),
                           preferred_element_type=jnp.float32)
    seg2 = lax.dot_general(hot2, segt_col, (((1,), (0,)), ((), ())),
                           preferred_element_type=jnp.float32)
    pos1_ref[...] = (rank1 + TILE * seg1).astype(jnp.int32)
    pos2_ref[...] = (rank2 + TILE * seg2).astype(jnp.int32)

    gi = lax.broadcasted_iota(jnp.int32, (E, G), 1).astype(jnp.float32)
    ei = lax.broadcasted_iota(jnp.int32, (E, G), 0).astype(jnp.float32)
    covg = jnp.logical_and(segt_col <= gi, gi < segt_col + ntiles_col
                           ).astype(jnp.float32)
    texp = jnp.sum(ei * covg, axis=0, keepdims=True)
    cover = jnp.sum(covg, axis=0, keepdims=True)
    le = jnp.max((lax.broadcasted_iota(jnp.int32, (E, 1), 0
                                       ).astype(jnp.float32) + 1.0)
                 * (ntiles_col > 0), axis=0, keepdims=True) - 1.0
    texp = texp + (1.0 - cover) * le
    total = jnp.sum(ntiles_col, axis=0, keepdims=True)
    sp_ref[...] = jnp.concatenate(
        [total.astype(jnp.int32), texp.astype(jnp.int32)], axis=1)

    f = counts_row * (1.0 / (T * K))
    p_mean = jnp.mean(probs, axis=0, keepdims=True)
    aux_ref[...] = E * jnp.sum(f * p_mean, keepdims=True).reshape(1, 1)


IR = 16
IC_ = T // IR
NCH = 4
RCH = RPT // NCH


def _dispatch_body(x_hbm, pos1_hbm, pos2_hbm, wn1_hbm, wn2_hbm, sp_hbm,
                   xs_hbm, sw_hbm,
                   p1row, p2row, w1row, w2row, tokrow, zrow, myidx,
                   rows_a, spv, shidx, gsa, gsb, wsa, wsb):
    c = lax.axis_index("c")
    s = lax.axis_index("s")
    wid = c * NS + s

    zi = jnp.zeros((16,), jnp.int32)

    def zero(i, acc):
        zrow[pl.ds(i * 16, 16)] = zi
        return acc
    lax.fori_loop(0, (S // NS) // 16, zero, 0)
    pltpu.sync_copy(zrow, shidx.at[pl.ds(s * (S // NS), S // NS)])
    pltpu.sync_copy(pos1_hbm.at[s], p1row)
    pltpu.sync_copy(pos2_hbm.at[s], p2row)
    pltpu.sync_copy(wn1_hbm.at[s], w1row)
    pltpu.sync_copy(wn2_hbm.at[s], w2row)
    pltpu.sync_copy(sp_hbm.at[pl.ds(0, 16)], spv)

    def mktok(i, acc):
        tokrow[pl.ds(i * 16, 16)] = (lax.iota(jnp.int32, 16)
                                     + (s * IC_ + i * 16))
        return acc
    lax.fori_loop(0, IC_ // 16, mktok, 0)
    plsc.subcore_barrier()
    descs = [
        pltpu.async_copy(tokrow, shidx.at[p1row], gsa),
        pltpu.async_copy(tokrow, shidx.at[p2row], gsb),
        pltpu.async_copy(w1row, sw_hbm.at[p1row], wsa),
        pltpu.async_copy(w2row, sw_hbm.at[p2row], wsb),
    ]
    for d in descs:
        d.wait()
    plsc.subcore_barrier()

    pltpu.sync_copy(shidx.at[pl.ds(wid * RPT, RPT)], myidx)
    total_rows = TILE * spv[...][0]
    nch = jnp.clip((total_rows - wid * RPT + RCH - 1) // RCH, 0, NCH)

    def chunk(ch, acc):
        pltpu.async_copy(x_hbm.at[myidx.at[pl.ds(ch * RCH, RCH)]],
                         rows_a, gsa).wait()
        pltpu.sync_copy(rows_a, xs_hbm.at[pl.ds(wid * RPT + ch * RCH, RCH)])
        return acc
    lax.fori_loop(0, nch, chunk, 0)


def _grouped_body(sp_ref, x_ref, wu_ref, wd_ref, w_ref, o_ref):
    ic = pl.program_id(0)
    g = pl.program_id(1)
    valid = g < sp_ref[0]

    @pl.when(valid)
    def _compute():
        xb = x_ref[...].astype(jnp.bfloat16)
        h = lax.dot_general(xb, wu_ref[0].astype(jnp.bfloat16),
                            (((1,), (1,)), ((), ())),
                            preferred_element_type=jnp.float32)
        h = _gelu_erf(h)
        y = lax.dot_general(h.astype(jnp.bfloat16),
                            wd_ref[0].astype(jnp.bfloat16),
                            (((1,), (1,)), ((), ())),
                            preferred_element_type=jnp.float32)
        y = w_ref[...] * y

        @pl.when(ic == 0)
        def _set():
            o_ref[pl.ds(g * TILE, TILE), :] = y

        @pl.when(ic > 0)
        def _acc():
            o_ref[pl.ds(g * TILE, TILE), :] += y


def _combine_body(y_hbm, pos1_hbm, pos2_hbm, out_hbm,
                  i1v, i2v, r1, r2, sem):
    c = lax.axis_index("c")
    s = lax.axis_index("s")
    wid = c * NS + s
    base = wid * TPT
    pltpu.sync_copy(pos1_hbm.at[pl.ds(base, TPT)], i1v)
    pltpu.sync_copy(pos2_hbm.at[pl.ds(base, TPT)], i2v)
    half = TPT // 2
    for ch in range(2):
        a = pltpu.async_copy(y_hbm.at[i1v.at[pl.ds(ch * half, half)]],
                             r1, sem)
        b = pltpu.async_copy(y_hbm.at[i2v.at[pl.ds(ch * half, half)]],
                             r2, sem)
        a.wait()
        b.wait()

        def add_tok(t, acc):
            for j in range(H // 16):
                sl = pl.ds(j * 16, 16)
                r1[t, sl] = r1[t, sl] + r2[t, sl]
            return acc
        lax.fori_loop(0, half, add_tok, 0)
        pltpu.sync_copy(r1, out_hbm.at[pl.ds(base + ch * half, half)])


@jax.jit
def kernel(hidden_states, W_router, W_up, W_down):
    batch, seq_len, hidden = hidden_states.shape
    flat = hidden_states.reshape(-1, hidden)

    sp, pos1, pos2, wn1, wn2, aux = pl.pallas_call(
        _router_body,
        out_shape=[
            jax.ShapeDtypeStruct((1, G + 1), jnp.int32),
            jax.ShapeDtypeStruct((T, 1), jnp.int32),
            jax.ShapeDtypeStruct((T, 1), jnp.int32),
            jax.ShapeDtypeStruct((T, 1), jnp.float32),
            jax.ShapeDtypeStruct((T, 1), jnp.float32),
            jax.ShapeDtypeStruct((1, 1), jnp.float32),
        ],
    )(flat, W_router)

    pos1f = pos1.reshape(T)
    pos2f = pos2.reshape(T)

    mesh = plsc.VectorSubcoreMesh(core_axis_name="c", subcore_axis_name="s")
    x_sorted, slot_w = pl.kernel(
        _dispatch_body,
        out_type=[
            jax.ShapeDtypeStruct((S, H), jnp.float32),
            jax.ShapeDtypeStruct((S,), jnp.float32),
        ],
        mesh=mesh,
        scratch_types=[
            pltpu.VMEM((IC_,), jnp.int32),
            pltpu.VMEM((IC_,), jnp.int32),
            pltpu.VMEM((IC_,), jnp.float32),
            pltpu.VMEM((IC_,), jnp.float32),
            pltpu.VMEM((IC_,), jnp.int32),
            pltpu.VMEM((S // NS,), jnp.int32),
            pltpu.VMEM((RPT,), jnp.int32),
            pltpu.VMEM((RCH, H), jnp.float32),
            pltpu.VMEM((16,), jnp.int32),
            pltpu.VMEM_SHARED((S,), jnp.int32),
            pltpu.SemaphoreType.DMA,
            pltpu.SemaphoreType.DMA,
            pltpu.SemaphoreType.DMA,
            pltpu.SemaphoreType.DMA,
        ],
    )(flat, pos1f.reshape(IR, IC_), pos2f.reshape(IR, IC_),
      wn1.reshape(IR, IC_), wn2.reshape(IR, IC_), sp.reshape(G + 1))

    y_sorted = pl.pallas_call(
        _grouped_body,
        grid_spec=pltpu.PrefetchScalarGridSpec(
            num_scalar_prefetch=1,
            grid=(I // TILE_I, G),
            in_specs=[
                pl.BlockSpec((TILE, H), lambda ic, g, sp: (g, 0)),
                pl.BlockSpec((1, TILE_I, H), lambda ic, g, sp: (sp[1 + g], ic, 0)),
                pl.BlockSpec((1, H, TILE_I), lambda ic, g, sp: (sp[1 + g], 0, ic)),
                pl.BlockSpec((TILE, 1), lambda ic, g, sp: (g, 0)),
            ],
            out_specs=pl.BlockSpec((S, H), lambda ic, g, sp: (0, 0)),
        ),
        out_shape=jax.ShapeDtypeStruct((S, H), jnp.float32),
        compiler_params=pltpu.CompilerParams(
            dimension_semantics=("arbitrary", "arbitrary"),
            vmem_limit_bytes=63 * 1024 * 1024),
    )(sp.reshape(G + 1), x_sorted, W_up, W_down, slot_w.reshape(S, 1))

    out = pl.kernel(
        _combine_body,
        out_type=jax.ShapeDtypeStruct((T, H), jnp.float32),
        mesh=plsc.VectorSubcoreMesh(core_axis_name="c", subcore_axis_name="s"),
        scratch_types=[
            pltpu.VMEM((TPT,), jnp.int32),
            pltpu.VMEM((TPT,), jnp.int32),
            pltpu.VMEM((TPT // 2, H), jnp.float32),
            pltpu.VMEM((TPT // 2, H), jnp.float32),
            pltpu.SemaphoreType.DMA,
        ],
    )(y_sorted, pos1f, pos2f)

    return out.reshape(batch, seq_len, hidden), aux.reshape(())
